# Initial kernel scaffold; baseline (speedup 1.0000x reference)
import jax
import jax.numpy as jnp
from jax.experimental import pallas as pl


def kernel(embeds, edge_weight, edge_index, u_idx, v_idx):
    raise NotImplementedError("write your pallas kernel here")


# speedup vs baseline: 1.9719x; 1.9719x over previous
import functools

import jax
import jax.numpy as jnp
from jax import lax
from jax.experimental import pallas as pl
from jax.experimental.pallas import tpu as pltpu
from jax.experimental.pallas import tpu_sc as plsc

_N_NODES = 10000
_D = 128
_NJ = _D // 16
_NW = 32
_RPT = 320
_N_PAD = _NW * _RPT
_EBLK = 128
_PPT = 512

_mesh = plsc.VectorSubcoreMesh(core_axis_name="c", subcore_axis_name="s")


def _layer_body(h_hbm, src_hbm, dl_hbm, w_hbm, bounds_hbm, acc_hbm,
                hnext_hbm, accout_hbm,
                bounds_v, idx_v, dl_v, w_v, rows_v, hloc, accv, sem):
    c = lax.axis_index("c")
    s = lax.axis_index("s")
    wid = s * 2 + c
    base = wid * _RPT

    pltpu.sync_copy(bounds_hbm, bounds_v)
    bv = bounds_v[pl.ds(wid, 16)]
    e_lo = bv[0]
    e_hi = bv[1]
    e_lo_al = (e_lo // 8) * 8
    nblk = (e_hi - e_lo_al + _EBLK - 1) // _EBLK

    zeros16 = jnp.zeros((16,), jnp.float32)
    lane = lax.iota(jnp.int32, 16)

    def zero_row(r, carry):
        for j in range(_NJ):
            hloc[r, pl.ds(j * 16, 16)] = zeros16
        return carry

    lax.fori_loop(0, _RPT, zero_row, 0)

    def blk(i, carry):
        bs = e_lo_al + i * _EBLK
        pltpu.sync_copy(src_hbm.at[pl.ds(bs, _EBLK)], idx_v)
        pltpu.sync_copy(dl_hbm.at[pl.ds(bs, _EBLK)], dl_v)
        pltpu.sync_copy(w_hbm.at[pl.ds(bs, _EBLK)], w_v)
        pltpu.async_copy(h_hbm.at[idx_v], rows_v, sem).wait()

        def edge16(q, c2):
            eb = q * 16
            egv = bs + eb + lane
            okv = (egv >= e_lo) & (egv < e_hi)
            w16 = jnp.where(okv, w_v[pl.ds(eb, 16)], 0.0)
            dl16 = dl_v[pl.ds(eb, 16)]
            for l in range(16):
                w = w16[l]
                dl = dl16[l]
                for j in range(_NJ):
                    val = rows_v[eb + l, pl.ds(j * 16, 16)] * w
                    plsc.addupdate(hloc.at[dl, pl.ds(j * 16, 16)], val)
            return c2

        lax.fori_loop(0, _EBLK // 16, edge16, 0)
        return carry

    lax.fori_loop(0, nblk, blk, 0)

    pltpu.sync_copy(acc_hbm.at[pl.ds(base, _RPT)], accv)

    def add_row(r, carry):
        for j in range(_NJ):
            sl = pl.ds(j * 16, 16)
            accv[r, sl] = accv[r, sl] + hloc[r, sl]
        return carry

    lax.fori_loop(0, _RPT, add_row, 0)

    pltpu.sync_copy(hloc, hnext_hbm.at[pl.ds(base, _RPT)])
    pltpu.sync_copy(accv, accout_hbm.at[pl.ds(base, _RPT)])


def _score_body(acc_hbm, u_hbm, v_hbm,
                out_hbm,
                uv, vv, urows, vrows, outv, sem):
    c = lax.axis_index("c")
    s = lax.axis_index("s")
    wid = s * 2 + c
    base = wid * _PPT
    lane = lax.iota(jnp.int32, 16)
    zeros16 = jnp.zeros((16,), jnp.float32)

    for k in range(_PPT // _EBLK):
        pltpu.sync_copy(u_hbm.at[pl.ds(base + k * _EBLK, _EBLK)], uv)
        pltpu.sync_copy(v_hbm.at[pl.ds(base + k * _EBLK, _EBLK)], vv)
        pltpu.async_copy(acc_hbm.at[uv], urows, sem).wait()
        pltpu.async_copy(acc_hbm.at[vv], vrows, sem).wait()

        def grp(g, carry):
            acc16 = zeros16
            for l in range(16):
                p = g * 16 + l
                t = urows[p, pl.ds(0, 16)] * vrows[p, pl.ds(0, 16)]
                for j in range(1, _NJ):
                    sl = pl.ds(j * 16, 16)
                    t = t + urows[p, sl] * vrows[p, sl]
                acc16 = jnp.where(lane == l, jnp.sum(t), acc16)
            x = acc16 * (1.0 / 16.0)
            outv[pl.ds(k * _EBLK + g * 16, 16)] = (
                1.0 + 4.0 / (1.0 + jnp.exp(-x)))
            return carry

        lax.fori_loop(0, _EBLK // 16, grp, 0)

    pltpu.sync_copy(outv, out_hbm.at[pl.ds(base, _PPT)])


@functools.partial(
    pl.kernel,
    out_type=[jax.ShapeDtypeStruct((_N_PAD, _D), jnp.float32),
              jax.ShapeDtypeStruct((_N_PAD, _D), jnp.float32)],
    mesh=_mesh,
    compiler_params=pltpu.CompilerParams(needs_layout_passes=False),
    scratch_types=[
        pltpu.VMEM((48,), jnp.int32),
        pltpu.VMEM((_EBLK,), jnp.int32),
        pltpu.VMEM((_EBLK,), jnp.int32),
        pltpu.VMEM((_EBLK,), jnp.float32),
        pltpu.VMEM((_EBLK, _D), jnp.float32),
        pltpu.VMEM((_RPT, _D), jnp.float32),
        pltpu.VMEM((_RPT, _D), jnp.float32),
        pltpu.SemaphoreType.DMA,
    ],
)
def _layer(*refs):
    _layer_body(*refs)


@functools.partial(
    pl.kernel,
    out_type=jax.ShapeDtypeStruct((_NW * _PPT,), jnp.float32),
    mesh=_mesh,
    compiler_params=pltpu.CompilerParams(needs_layout_passes=False),
    scratch_types=[
        pltpu.VMEM((_EBLK,), jnp.int32),
        pltpu.VMEM((_EBLK,), jnp.int32),
        pltpu.VMEM((_EBLK, _D), jnp.float32),
        pltpu.VMEM((_EBLK, _D), jnp.float32),
        pltpu.VMEM((_PPT,), jnp.float32),
        pltpu.SemaphoreType.DMA,
    ],
)
def _score(*refs):
    _score_body(*refs)


def kernel(embeds, edge_weight, edge_index, u_idx, v_idx):
    dst = edge_index[0]
    src = edge_index[1]
    dst_s, src_s, w_s = lax.sort((dst, src, edge_weight), num_keys=1)
    dl_s = dst_s - (dst_s // _RPT) * _RPT
    n_edges = dst.shape[0]
    ep = n_edges + 2 * _EBLK
    src_p = jnp.concatenate([src_s, jnp.zeros((ep - n_edges,), jnp.int32)])
    dl_p = jnp.concatenate([dl_s, jnp.zeros((ep - n_edges,), jnp.int32)])
    w_p = jnp.concatenate([w_s, jnp.zeros((ep - n_edges,), jnp.float32)])
    tile_starts = jnp.arange(0, _N_PAD + 1, _RPT, dtype=jnp.int32)
    bounds = jnp.searchsorted(dst_s, tile_starts).astype(jnp.int32)
    bounds = jnp.concatenate([bounds, jnp.zeros((15,), jnp.int32)])
    h0 = jnp.pad(embeds, ((0, _N_PAD - embeds.shape[0]), (0, 0)))

    h1, a1 = _layer(h0, src_p, dl_p, w_p, bounds, h0)
    h2, a2 = _layer(h1, src_p, dl_p, w_p, bounds, a1)
    _, a3 = _layer(h2, src_p, dl_p, w_p, bounds, a2)
    return _score(a3, u_idx, v_idx)
